# baseline (speedup 1.0000x reference)
import functools

import jax
import jax.numpy as jnp
from jax import lax
from jax.experimental import pallas as pl
from jax.experimental.pallas import tpu as pltpu
from jax.experimental.pallas import tpu_sc as plsc

N = 10000
E = 320000
D = 128
DH = D // 2
NC = 2
NS = 16
K = 80
CH = 50
NCH = 5
NWIN_T = NCH * CH
NROWS_T = N // NS
NEG_SLOPE = 0.2


def _tc_prep(x, weight, a2):
    def body(x_ref, w_ref, a2_ref, h2_ref, aN_ref):
        xb = x_ref[...]
        hb = jnp.dot(xb, w_ref[...], preferred_element_type=jnp.float32)
        h2_ref[0] = hb[:, :DH]
        h2_ref[1] = hb[:, DH:]
        aN_ref[...] = lax.dot_general(
            hb, a2_ref[...], (((1,), (1,)), ((), ())),
            preferred_element_type=jnp.float32)

    blk = 1000
    return pl.pallas_call(
        body,
        grid=(N // blk,),
        in_specs=[
            pl.BlockSpec((blk, D), lambda i: (i, 0)),
            pl.BlockSpec((D, D), lambda i: (0, 0)),
            pl.BlockSpec((2, D), lambda i: (0, 0)),
        ],
        out_specs=[
            pl.BlockSpec((2, blk, DH), lambda i: (0, i, 0)),
            pl.BlockSpec((blk, 2), lambda i: (i, 0)),
        ],
        out_shape=[
            jax.ShapeDtypeStruct((2, N, DH), jnp.float32),
            jax.ShapeDtypeStruct((N, 2), jnp.float32),
        ],
    )(x, weight, a2)


def _sc_main(h2, aflat, src4, dst4):
    mesh = plsc.VectorSubcoreMesh(core_axis_name="c", subcore_axis_name="s")

    @functools.partial(
        pl.kernel,
        mesh=mesh,
        compiler_params=pltpu.CompilerParams(
            needs_layout_passes=False, use_tc_tiling_on_sc=False),
        out_type=jax.ShapeDtypeStruct((NC, NS, NROWS_T, DH), jnp.float32),
        scratch_types=[
            pltpu.VMEM((CH, K), jnp.int32),
            pltpu.VMEM((CH, K), jnp.int32),
            pltpu.VMEM((2 * N,), jnp.float32),
            pltpu.VMEM((NWIN_T * K,), jnp.float32),
            pltpu.VMEM((N,), jnp.float32),
            pltpu.VMEM((K, DH), jnp.float32),
            pltpu.VMEM((K, DH), jnp.float32),
            pltpu.VMEM((K, DH), jnp.float32),
            pltpu.VMEM((K, DH), jnp.float32),
            pltpu.VMEM((640,), jnp.float32),
            pltpu.VMEM_SHARED((N,), jnp.float32),
            pltpu.VMEM_SHARED((N, DH), jnp.float32),
            pltpu.SemaphoreType.DMA,
            pltpu.SemaphoreType.DMA,
            pltpu.SemaphoreType.DMA,
            pltpu.SemaphoreType.DMA,
            pltpu.SemaphoreType.DMA,
            pltpu.SemaphoreType.DMA,
        ],
    )
    def k(h2_hbm, a2_hbm, src_hbm, dst_hbm, out_hbm,
          dst_ch, src_ch, a2_loc, ex_all, denom_loc, g_a, g_b, s_a, s_b,
          zbuf, denom_sh, acc_sh,
          sem_ga, sem_gb, sem_sa, sem_sb, sem_p, sem_z):
        c = lax.axis_index("c")
        s = lax.axis_index("s")
        h_hbm = h2_hbm.at[c]

        pltpu.sync_copy(a2_hbm, a2_loc)

        z16 = jnp.zeros((16,), jnp.float32)

        def zz(i, carry):
            zbuf[pl.ds(i * 16, 16)] = z16
            return carry

        lax.fori_loop(0, 640 // 16, zz, 0)

        def zrow(r, carry):
            for j in range(DH // 16):
                s_a[r, pl.ds(j * 16, 16)] = z16
            return carry

        lax.fori_loop(0, K, zrow, 0)

        row0 = s * NROWS_T
        n_full = NROWS_T // K
        rem = NROWS_T - n_full * K
        zh = []
        for kk in range(n_full):
            zh.append(pltpu.async_copy(
                s_a, acc_sh.at[pl.ds(row0 + kk * K, K), :], sem_z))
        pltpu.sync_copy(s_a.at[pl.ds(0, rem), :],
                        acc_sh.at[pl.ds(row0 + n_full * K, rem), :])

        @pl.when(s == 0)
        def _():
            def zd(i, carry):
                pltpu.sync_copy(zbuf, denom_sh.at[pl.ds(i * 640, 640)])
                return carry

            lax.fori_loop(0, N // 640, zd, 0)
            pltpu.sync_copy(zbuf.at[pl.ds(0, N - (N // 640) * 640)],
                            denom_sh.at[pl.ds((N // 640) * 640,
                                              N - (N // 640) * 640)])

        for h_ in zh:
            h_.wait()
        plsc.subcore_barrier()

        def p1(ch, carry):
            pltpu.sync_copy(src_hbm.at[s, ch], src_ch)
            pltpu.sync_copy(dst_hbm.at[s, ch], dst_ch)
            base_w = ch * CH

            def win(cb, wcarry):
                ebase = (base_w + cb) * K
                for q in range(K // 16):
                    d16 = dst_ch[cb, pl.ds(q * 16, 16)]
                    s16 = src_ch[cb, pl.ds(q * 16, 16)]
                    ad = plsc.load_gather(a2_loc, [d16 * 2])
                    asv = plsc.load_gather(a2_loc, [s16 * 2 + 1])
                    al = ad + asv
                    al = jnp.where(al >= 0.0, al, NEG_SLOPE * al)
                    ex_all[pl.ds(ebase + q * 16, 16)] = jnp.exp(al)

                @pl.when(cb > 0)
                def _():
                    pltpu.make_async_copy(
                        ex_all.at[pl.ds(0, K)],
                        denom_sh.at[dst_ch.at[0]], sem_p).wait()

                pltpu.async_copy(ex_all.at[pl.ds(ebase, K)],
                                 denom_sh.at[dst_ch.at[cb]], sem_p, add=True)
                return wcarry

            lax.fori_loop(0, CH, win, 0)
            pltpu.make_async_copy(
                ex_all.at[pl.ds(0, K)], denom_sh.at[dst_ch.at[0]],
                sem_p).wait()
            return carry

        lax.fori_loop(0, NCH, p1, 0)

        plsc.subcore_barrier()
        pltpu.sync_copy(denom_sh, denom_loc)

        def mult(g_ref, s_ref, wt):
            def rmul(i, rcarry):
                for u in range(4):
                    r = i * 4 + u
                    c16 = plsc.load_gather(
                        ex_all, [jnp.full((16,), wt * K + r, jnp.int32)])
                    for j in range(DH // 16):
                        s_ref[r, pl.ds(j * 16, 16)] = (
                            c16 * g_ref[r, pl.ds(j * 16, 16)])
                return rcarry

            lax.fori_loop(0, K // 4, rmul, 0)

        def p2(ch, carry):
            pltpu.sync_copy(src_hbm.at[s, ch], src_ch)
            pltpu.sync_copy(dst_hbm.at[s, ch], dst_ch)
            base_w = ch * CH

            def pha(cb, wcarry):
                ebase = (base_w + cb) * K
                for q in range(K // 16):
                    d16 = dst_ch[cb, pl.ds(q * 16, 16)]
                    den16 = plsc.load_gather(denom_loc, [d16])
                    ex16 = ex_all[pl.ds(ebase + q * 16, 16)]
                    ex_all[pl.ds(ebase + q * 16, 16)] = (
                        ex16 / (den16 + 1e-16))
                return wcarry

            lax.fori_loop(0, CH, pha, 0)

            pltpu.async_copy(h_hbm.at[src_ch.at[0]], g_a, sem_ga)
            pltpu.async_copy(h_hbm.at[src_ch.at[1]], g_b, sem_gb)

            def pair(g, wcarry):
                wa = 2 * g
                wb = wa + 1

                pltpu.make_async_copy(
                    h_hbm.at[pl.ds(0, K)], g_a, sem_ga).wait()

                @pl.when(g > 0)
                def _():
                    pltpu.make_async_copy(
                        s_a, acc_sh.at[dst_ch.at[0]], sem_sa).wait()

                mult(g_a, s_a, base_w + wa)

                @pl.when(wa + 2 < CH)
                def _():
                    pltpu.async_copy(
                        h_hbm.at[src_ch.at[wa + 2]], g_a, sem_ga)

                pltpu.async_copy(s_a, acc_sh.at[dst_ch.at[wa]],
                                 sem_sa, add=True)

                pltpu.make_async_copy(
                    h_hbm.at[pl.ds(0, K)], g_b, sem_gb).wait()

                @pl.when(g > 0)
                def _():
                    pltpu.make_async_copy(
                        s_b, acc_sh.at[dst_ch.at[0]], sem_sb).wait()

                mult(g_b, s_b, base_w + wb)

                @pl.when(wb + 2 < CH)
                def _():
                    pltpu.async_copy(
                        h_hbm.at[src_ch.at[wb + 2]], g_b, sem_gb)

                pltpu.async_copy(s_b, acc_sh.at[dst_ch.at[wb]],
                                 sem_sb, add=True)
                return wcarry

            lax.fori_loop(0, CH // 2, pair, 0)

            pltpu.make_async_copy(
                s_a, acc_sh.at[dst_ch.at[0]], sem_sa).wait()
            pltpu.make_async_copy(
                s_b, acc_sh.at[dst_ch.at[0]], sem_sb).wait()
            return carry

        lax.fori_loop(0, NCH, p2, 0)

        plsc.subcore_barrier()
        pltpu.sync_copy(acc_sh.at[pl.ds(row0, NROWS_T), :],
                        out_hbm.at[c, s])

    return k(h2, aflat, src4, dst4)


def _tc_epilogue(partials, bias2):
    def body(p_ref, b_ref, o_ref):
        o_ref[...] = (
            jnp.concatenate([p_ref[0], p_ref[1]], axis=-1) + b_ref[...])

    blk = 1000
    return pl.pallas_call(
        body,
        grid=(N // blk,),
        in_specs=[
            pl.BlockSpec((NC, blk, DH), lambda i: (0, i, 0)),
            pl.BlockSpec((1, D), lambda i: (0, 0)),
        ],
        out_specs=pl.BlockSpec((blk, D), lambda i: (i, 0)),
        out_shape=jax.ShapeDtypeStruct((N, D), jnp.float32),
    )(partials, bias2)


def kernel(x, edge_index, weight, att, bias):
    ei = edge_index.astype(jnp.int32)
    src4 = ei[0].reshape(NS, NCH, CH, K)
    dst4 = ei[1].reshape(NS, NCH, CH, K)
    a2 = att.reshape(2, D)
    h2, aN = _tc_prep(x, weight, a2)
    partials = _sc_main(h2, aN.reshape(2 * N), src4, dst4)
    partials = partials.reshape(NC, N, DH)
    return _tc_epilogue(partials, bias.reshape(1, D))
